# Initial kernel scaffold; baseline (speedup 1.0000x reference)
import jax
import jax.numpy as jnp
from jax.experimental import pallas as pl


def kernel(tokenized_prompts, token_embedding):
    raise NotImplementedError("write your pallas kernel here")


# speedup vs baseline: 1.1683x; 1.1683x over previous
import functools

import jax
import jax.numpy as jnp
from jax import lax
from jax.experimental import pallas as pl
from jax.experimental.pallas import tpu as pltpu
from jax.experimental.pallas import tpu_sc as plsc

N_CLASSES = 1000
CTX_LEN = 77
DIM = 512
B = N_CLASSES * CTX_LEN

CS = 88
NCHUNKS = B // CS
NW = 32
MAXCH = -(-NCHUNKS // NW)

_mesh = plsc.VectorSubcoreMesh(core_axis_name="c", subcore_axis_name="s")


@functools.partial(
    pl.kernel,
    mesh=_mesh,
    out_type=jax.ShapeDtypeStruct((B, DIM), jnp.float32),
    scratch_types=[
        pltpu.VMEM((CS,), jnp.int32),
        pltpu.VMEM((CS, DIM), jnp.float32),
        pltpu.SemaphoreType.DMA,
    ],
)
def _gather_kernel(idx_hbm, table_hbm, out_hbm, idx_v, rows_v, sem):
    wid = lax.axis_index("s") * 2 + lax.axis_index("c")

    def body(j, carry):
        c = wid + NW * j

        @pl.when(c < NCHUNKS)
        def _():
            base = c * CS
            pltpu.sync_copy(idx_hbm.at[pl.ds(base, CS)], idx_v)
            pltpu.async_copy(table_hbm.at[idx_v], rows_v, sem).wait()
            pltpu.sync_copy(rows_v, out_hbm.at[pl.ds(base, CS)])

        return carry

    lax.fori_loop(0, MAXCH, body, 0)


def kernel(tokenized_prompts, token_embedding):
    flat_idx = tokenized_prompts.reshape(B)
    out = _gather_kernel(flat_idx, token_embedding)
    return out.reshape(N_CLASSES, CTX_LEN, DIM)


# speedup vs baseline: 1.2012x; 1.0282x over previous
import functools

import jax
import jax.numpy as jnp
from jax import lax
from jax.experimental import pallas as pl
from jax.experimental.pallas import tpu as pltpu
from jax.experimental.pallas import tpu_sc as plsc

N_CLASSES = 1000
CTX_LEN = 77
DIM = 512
B = N_CLASSES * CTX_LEN

CS = 56
NCHUNKS = B // CS
NW = 32
MAIN = 42
TAIL = NCHUNKS - NW * MAIN
NBUF = 3
ROUNDS = MAIN // NBUF

_mesh = plsc.VectorSubcoreMesh(core_axis_name="c", subcore_axis_name="s")


@functools.partial(
    pl.kernel,
    mesh=_mesh,
    out_type=jax.ShapeDtypeStruct((B, DIM), jnp.float32),
    scratch_types=[
        pltpu.VMEM(((MAIN + 1) * CS,), jnp.int32),
        pltpu.VMEM((NBUF, CS, DIM), jnp.float32),
        pltpu.SemaphoreType.DMA,
        pltpu.SemaphoreType.DMA,
        pltpu.SemaphoreType.DMA,
    ],
)
def _gather_kernel(idx_hbm, table_hbm, out_hbm, idx_v, rows_v, sem0, sem1, sem2):
    wid = lax.axis_index("s") * 2 + lax.axis_index("c")
    sems = (sem0, sem1, sem2)
    c0 = wid * MAIN

    pltpu.sync_copy(idx_hbm.at[pl.ds(c0 * CS, MAIN * CS)], idx_v.at[pl.ds(0, MAIN * CS)])

    has_tail = wid < TAIL
    tail_c = NW * MAIN + wid

    @pl.when(has_tail)
    def _():
        pltpu.sync_copy(idx_hbm.at[pl.ds(tail_c * CS, CS)], idx_v.at[pl.ds(MAIN * CS, CS)])

    def gather(j, b):
        pltpu.async_copy(table_hbm.at[idx_v.at[pl.ds(j * CS, CS)]], rows_v.at[b], sems[b])

    def wait(b):
        pltpu.make_async_copy(table_hbm.at[pl.ds(0, CS)], rows_v.at[b], sems[b]).wait()

    def store(c, b):
        pltpu.async_copy(rows_v.at[b], out_hbm.at[pl.ds(c * CS, CS)], sems[b])

    def round_body(i, carry):
        g = i * NBUF
        for b in range(NBUF):
            @pl.when(i > 0)
            def _():
                wait(b)
            gather(g + b, b)
        for b in range(NBUF):
            wait(b)
            store(c0 + g + b, b)
        return carry

    lax.fori_loop(0, ROUNDS, round_body, 0)

    wait(0)

    @pl.when(has_tail)
    def _():
        gather(MAIN, 0)
        wait(0)
        store(tail_c, 0)
        wait(0)

    wait(1)
    wait(2)


def kernel(tokenized_prompts, token_embedding):
    flat_idx = tokenized_prompts.reshape(B)
    out = _gather_kernel(flat_idx, token_embedding)
    return out.reshape(N_CLASSES, CTX_LEN, DIM)
